# initial kernel scaffold (unmeasured)
import jax
import jax.numpy as jnp
from jax import lax
from jax.experimental import pallas as pl
from jax.experimental.pallas import tpu as pltpu

N_DEV = 4


def kernel(x, w_mat, scale_x, scale_w):
    m_total, k_per = x.shape
    k_total, n = w_mat.shape
    m_per = m_total // N_DEV
    assert k_total == N_DEV * k_per

    NT = 2048
    n_tiles = n // NT

    x8 = x.astype(jnp.float8_e4m3fn)
    w8 = w_mat.astype(jnp.float8_e4m3fn)

    def body(x_ref, w_ref, sx_ref, sw_ref, out_ref, xg_ref, send_sems, recv_sems):
        my = lax.axis_index("i")

        barrier = pltpu.get_barrier_semaphore()
        for off in range(1, N_DEV):
            pl.semaphore_signal(
                barrier, inc=1,
                device_id=((my + off) % N_DEV,),
                device_id_type=pl.DeviceIdType.MESH,
            )
        pl.semaphore_wait(barrier, N_DEV - 1)

        sends = []
        for off in range(1, N_DEV):
            dst = (my + off) % N_DEV
            rdma = pltpu.make_async_remote_copy(
                src_ref=x_ref.at[pl.ds(dst * m_per, m_per), :],
                dst_ref=xg_ref.at[my],
                send_sem=send_sems.at[off],
                recv_sem=recv_sems.at[my],
                device_id=(dst,),
                device_id_type=pl.DeviceIdType.MESH,
            )
            rdma.start()
            sends.append(rdma)

        def accum_block(p, block, first):
            for j in range(n_tiles):
                nd = pl.ds(j * NT, NT)
                w_tile = w_ref[pl.ds(p * k_per, k_per), nd]
                prod = jnp.dot(block, w_tile, preferred_element_type=jnp.float32)
                if first:
                    out_ref[:, nd] = prod
                else:
                    out_ref[:, nd] = out_ref[:, nd] + prod

        accum_block(my, x_ref[pl.ds(my * m_per, m_per), :], first=True)

        for off in range(1, N_DEV):
            src = (my - off) % N_DEV
            recv = pltpu.make_async_remote_copy(
                src_ref=x_ref.at[pl.ds(0, m_per), :],
                dst_ref=xg_ref.at[src],
                send_sem=send_sems.at[0],
                recv_sem=recv_sems.at[src],
                device_id=(my,),
                device_id_type=pl.DeviceIdType.MESH,
            )
            recv.wait_recv()
            accum_block(src, xg_ref[src], first=False)

        for rdma in sends:
            rdma.wait_send()

        s = sx_ref[0] * sw_ref[0]
        for j in range(n_tiles):
            nd = pl.ds(j * NT, NT)
            out_ref[:, nd] = out_ref[:, nd] * s

    return pl.pallas_call(
        body,
        out_shape=jax.ShapeDtypeStruct((m_per, n), jnp.float32),
        in_specs=[
            pl.BlockSpec(memory_space=pltpu.VMEM),
            pl.BlockSpec(memory_space=pltpu.VMEM),
            pl.BlockSpec(memory_space=pltpu.SMEM),
            pl.BlockSpec(memory_space=pltpu.SMEM),
        ],
        out_specs=pl.BlockSpec(memory_space=pltpu.VMEM),
        scratch_shapes=[
            pltpu.VMEM((N_DEV, m_per, k_per), jnp.float8_e4m3fn),
            pltpu.SemaphoreType.DMA((N_DEV,)),
            pltpu.SemaphoreType.DMA((N_DEV,)),
        ],
        compiler_params=pltpu.CompilerParams(
            collective_id=0,
            vmem_limit_bytes=128 * 1024 * 1024,
        ),
    )(x8, w8, scale_x, scale_w)


# baseline (device time: 143731 ns/iter reference)
import jax
import jax.numpy as jnp
from jax import lax
from jax.experimental import pallas as pl
from jax.experimental.pallas import tpu as pltpu

N_DEV = 4
NT = 2048


def kernel(x, w_mat, scale_x, scale_w):
    m_total, k_per = x.shape
    k_total, n = w_mat.shape
    m_per = m_total // N_DEV
    assert k_total == N_DEV * k_per
    n_tiles = n // NT

    x8 = x.astype(jnp.float8_e4m3fn)
    w8 = w_mat.astype(jnp.float8_e4m3fn)

    def body(x_ref, w_ref, sx_ref, sw_ref, out_ref,
             xg_ref, wt_ref, send_sems, recv_sems, w_sems, copy_sem):
        my = lax.axis_index("i")

        barrier = pltpu.get_barrier_semaphore()
        for off in range(1, N_DEV):
            pl.semaphore_signal(
                barrier, inc=1,
                device_id=((my + off) % N_DEV,),
                device_id_type=pl.DeviceIdType.MESH,
            )
        pl.semaphore_wait(barrier, N_DEV - 1)

        loc = pltpu.make_async_copy(
            x_ref.at[pl.ds(my * m_per, m_per), :], xg_ref.at[my], copy_sem)
        loc.start()

        sends = []
        for off in range(1, N_DEV):
            dst = (my + off) % N_DEV
            rdma = pltpu.make_async_remote_copy(
                src_ref=x_ref.at[pl.ds(dst * m_per, m_per), :],
                dst_ref=xg_ref.at[my],
                send_sem=send_sems.at[off],
                recv_sem=recv_sems.at[my],
                device_id=(dst,),
                device_id_type=pl.DeviceIdType.MESH,
            )
            rdma.start()
            sends.append(rdma)

        offs = [0, 1, 3, 2]
        srcs = [(my - off) % N_DEV for off in offs]
        tiles = [(pi, j) for pi in range(N_DEV) for j in range(n_tiles)]

        def start_w(t, slot):
            pi, j = tiles[t]
            cp = pltpu.make_async_copy(
                w_ref.at[pl.ds(srcs[pi] * k_per, k_per), pl.ds(j * NT, NT)],
                wt_ref.at[slot],
                w_sems.at[slot],
            )
            cp.start()
            return cp

        def wait_w(t, slot):
            pltpu.make_async_copy(
                w_ref.at[pl.ds(0, k_per), pl.ds(0, NT)],
                wt_ref.at[slot],
                w_sems.at[slot],
            ).wait()

        def wait_recv(src):
            pltpu.make_async_remote_copy(
                src_ref=x_ref.at[pl.ds(0, m_per), :],
                dst_ref=xg_ref.at[src],
                send_sem=send_sems.at[0],
                recv_sem=recv_sems.at[src],
                device_id=(my,),
                device_id_type=pl.DeviceIdType.MESH,
            ).wait_recv()

        s = sx_ref[0] * sw_ref[0]

        start_w(0, 0)
        loc.wait()
        for t, (pi, j) in enumerate(tiles):
            slot = t % 2
            if t + 1 < len(tiles):
                start_w(t + 1, (t + 1) % 2)
            if j == 0 and pi > 0:
                wait_recv(srcs[pi])
            wait_w(t, slot)
            block = xg_ref[pl.ds(srcs[pi], 1)].reshape(m_per, k_per)
            prod = jnp.dot(block, wt_ref[slot], preferred_element_type=jnp.float32)
            nd = pl.ds(j * NT, NT)
            if pi == 0:
                out_ref[:, nd] = prod
            elif pi == N_DEV - 1:
                out_ref[:, nd] = (out_ref[:, nd] + prod) * s
            else:
                out_ref[:, nd] = out_ref[:, nd] + prod

        for rdma in sends:
            rdma.wait_send()

    return pl.pallas_call(
        body,
        out_shape=jax.ShapeDtypeStruct((m_per, n), jnp.float32),
        in_specs=[
            pl.BlockSpec(memory_space=pltpu.VMEM),
            pl.BlockSpec(memory_space=pltpu.MemorySpace.HBM),
            pl.BlockSpec(memory_space=pltpu.SMEM),
            pl.BlockSpec(memory_space=pltpu.SMEM),
        ],
        out_specs=pl.BlockSpec(memory_space=pltpu.VMEM),
        scratch_shapes=[
            pltpu.VMEM((N_DEV, m_per, k_per), jnp.float8_e4m3fn),
            pltpu.VMEM((2, k_per, NT), jnp.float8_e4m3fn),
            pltpu.SemaphoreType.DMA((N_DEV,)),
            pltpu.SemaphoreType.DMA((N_DEV,)),
            pltpu.SemaphoreType.DMA((2,)),
            pltpu.SemaphoreType.DMA,
        ],
        compiler_params=pltpu.CompilerParams(
            collective_id=0,
            vmem_limit_bytes=100 * 1024 * 1024,
        ),
    )(x8, w8, scale_x, scale_w)


# device time: 104806 ns/iter; 1.3714x vs baseline; 1.3714x over previous
import jax
import jax.numpy as jnp
from jax import lax
from jax.experimental import pallas as pl
from jax.experimental.pallas import tpu as pltpu

N_DEV = 4
NT = 1024

_SLOT_OFF = [0, -1, 1, 2]
_SEND_SLOT = {1: 1, -1: 2, 2: 3}


def kernel(x, w_mat, scale_x, scale_w):
    m_total, k_per = x.shape
    k_total, n = w_mat.shape
    m_per = m_total // N_DEV
    assert k_total == N_DEV * k_per
    n_tiles = n // NT

    x8 = x.astype(jnp.float8_e4m3fn)

    def body(x_ref, w_ref, sx_ref, sw_ref, out_ref,
             xg_ref, wt_ref, acc_ref, send_sems, recv_sems, w_sems, copy_sem,
             out_sems):
        my = lax.axis_index("i")
        srcs = [(my + o) % N_DEV for o in _SLOT_OFF]

        barrier = pltpu.get_barrier_semaphore()
        for off in range(1, N_DEV):
            pl.semaphore_signal(
                barrier, inc=1,
                device_id=((my + off) % N_DEV,),
                device_id_type=pl.DeviceIdType.MESH,
            )
        pl.semaphore_wait(barrier, N_DEV - 1)

        loc = pltpu.make_async_copy(
            x_ref.at[pl.ds(my * m_per, m_per), :], xg_ref.at[0], copy_sem)
        loc.start()

        sends = []
        for off, slot in _SEND_SLOT.items():
            dst = (my + off) % N_DEV
            rdma = pltpu.make_async_remote_copy(
                src_ref=x_ref.at[pl.ds(dst * m_per, m_per), :],
                dst_ref=xg_ref.at[slot],
                send_sem=send_sems.at[slot],
                recv_sem=recv_sems.at[slot],
                device_id=(dst,),
                device_id_type=pl.DeviceIdType.MESH,
            )
            rdma.start()
            sends.append(rdma)

        tiles = [(pi, j) for pi in range(N_DEV) for j in range(n_tiles)]

        def start_w(t, slot):
            pi, j = tiles[t]
            pltpu.make_async_copy(
                w_ref.at[pl.ds(srcs[pi] * k_per, k_per), pl.ds(j * NT, NT)],
                wt_ref.at[slot],
                w_sems.at[slot],
            ).start()

        def wait_w(slot):
            pltpu.make_async_copy(
                w_ref.at[pl.ds(0, k_per), pl.ds(0, NT)],
                wt_ref.at[slot],
                w_sems.at[slot],
            ).wait()

        def wait_recv(slot):
            pltpu.make_async_remote_copy(
                src_ref=x_ref.at[pl.ds(0, m_per), :],
                dst_ref=xg_ref.at[slot],
                send_sem=send_sems.at[0],
                recv_sem=recv_sems.at[slot],
                device_id=(my,),
                device_id_type=pl.DeviceIdType.MESH,
            ).wait_recv()

        s = sx_ref[0] * sw_ref[0]

        start_w(0, 0)
        loc.wait()
        out_dmas = []
        for t, (pi, j) in enumerate(tiles):
            slot = t % 2
            if t + 1 < len(tiles):
                start_w(t + 1, (t + 1) % 2)
            if j == 0 and pi > 0:
                wait_recv(pi)
            wait_w(slot)
            w_tile = wt_ref[slot].astype(jnp.float8_e4m3fn)
            prod = jnp.dot(xg_ref[pi], w_tile,
                           preferred_element_type=jnp.float32)
            nd = pl.ds(j * NT, NT)
            if pi == 0:
                acc_ref[:, nd] = prod
            elif pi == N_DEV - 1:
                acc_ref[:, nd] = (acc_ref[:, nd] + prod) * s
                dma = pltpu.make_async_copy(
                    acc_ref.at[:, nd], out_ref.at[:, nd], out_sems.at[j])
                dma.start()
                out_dmas.append(dma)
            else:
                acc_ref[:, nd] = acc_ref[:, nd] + prod

        for rdma in sends:
            rdma.wait_send()
        for dma in out_dmas:
            dma.wait()

    return pl.pallas_call(
        body,
        out_shape=jax.ShapeDtypeStruct((m_per, n), jnp.float32),
        in_specs=[
            pl.BlockSpec(memory_space=pltpu.VMEM),
            pl.BlockSpec(memory_space=pltpu.MemorySpace.HBM),
            pl.BlockSpec(memory_space=pltpu.SMEM),
            pl.BlockSpec(memory_space=pltpu.SMEM),
        ],
        out_specs=pl.BlockSpec(memory_space=pltpu.MemorySpace.HBM),
        scratch_shapes=[
            pltpu.VMEM((N_DEV, m_per, k_per), jnp.float8_e4m3fn),
            pltpu.VMEM((2, k_per, NT), jnp.float32),
            pltpu.VMEM((m_per, n), jnp.float32),
            pltpu.SemaphoreType.DMA((N_DEV,)),
            pltpu.SemaphoreType.DMA((N_DEV,)),
            pltpu.SemaphoreType.DMA((2,)),
            pltpu.SemaphoreType.DMA,
            pltpu.SemaphoreType.DMA((n_tiles,)),
        ],
        compiler_params=pltpu.CompilerParams(
            collective_id=0,
            vmem_limit_bytes=100 * 1024 * 1024,
        ),
    )(x8, w_mat, scale_x, scale_w)
